# baseline (speedup 1.0000x reference)
import functools

import jax
import jax.numpy as jnp
from jax import lax
from jax.experimental import pallas as pl
from jax.experimental.pallas import tpu as pltpu
from jax.experimental.pallas import tpu_sc as plsc

N_NODES = 10000
D = 128
N_EDGES = 320000
MAX_DEG = 10
N_GRAPHS = 64

NC, NS, L = 2, 16, 16
NW = NC * NS
CHUNK = 128
CPT = 160
EDGES_PAD = NS * CPT * CHUNK
HD = D // NC
NPAD = 10016
ROWS_PT = NPAD // NS
CPT_CNT = CPT // NC

BM = 2000


NB = 8
PF = 3


def _make_sc_agg(with_count):
    mesh = plsc.VectorSubcoreMesh(
        core_axis_name="c", subcore_axis_name="s",
        num_cores=NC, num_subcores=NS)

    out_type = [jax.ShapeDtypeStruct((NC, NPAD, HD), jnp.bfloat16)]
    scratch = [
        pltpu.VMEM((CPT, CHUNK), jnp.int32),
        pltpu.VMEM((CPT, CHUNK), jnp.int32),
        [pltpu.VMEM((CHUNK, HD), jnp.bfloat16)] * NB,
        pltpu.VMEM_SHARED((NPAD, HD), jnp.bfloat16),
        [pltpu.SemaphoreType.DMA] * NB,
        [pltpu.SemaphoreType.DMA] * NB,
    ]
    if with_count:
        out_type.append(jax.ShapeDtypeStruct((NC, NPAD, L), jnp.int32))
        scratch.append(pltpu.VMEM((CHUNK, L), jnp.int32))
        scratch.append(pltpu.VMEM_SHARED((NPAD, L), jnp.int32))

    @functools.partial(
        pl.kernel,
        out_type=out_type,
        mesh=mesh,
        compiler_params=pltpu.CompilerParams(use_tc_tiling_on_sc=False),
        scratch_types=scratch,
    )
    def sc_agg(h_hbm, srcs_hbm, dsts_hbm, *refs):
        if with_count:
            (out_hbm, cnt_hbm, src_v, dst_v, bufs, acc, gsem, ssem,
             ones_v, cacc) = refs
        else:
            out_hbm, src_v, dst_v, bufs, acc, gsem, ssem = refs
        c = lax.axis_index("c")
        s = lax.axis_index("s")

        pltpu.sync_copy(srcs_hbm.at[c, s], src_v)
        pltpu.sync_copy(dsts_hbm.at[s], dst_v)

        zeros = jnp.zeros((2 * L,), jnp.bfloat16)

        def zrow(r, _):
            for cc in range(HD // (2 * L)):
                bufs[0][r, pl.ds(cc * 2 * L, 2 * L)] = zeros
            return 0

        lax.fori_loop(0, CHUNK, zrow, 0)
        for r in range(ROWS_PT // CHUNK + 1):
            n = min(CHUNK, ROWS_PT - r * CHUNK)
            if n > 0:
                pltpu.sync_copy(
                    bufs[0].at[pl.ds(0, n)],
                    acc.at[pl.ds(s * ROWS_PT + r * CHUNK, n)])

        if with_count:
            izeros = jnp.zeros((L,), jnp.int32)

            def czrow(r, _):
                ones_v[r, pl.ds(0, L)] = izeros
                return 0

            lax.fori_loop(0, CHUNK, czrow, 0)
            for r in range(ROWS_PT // CHUNK + 1):
                n = min(CHUNK, ROWS_PT - r * CHUNK)
                if n > 0:
                    pltpu.sync_copy(
                        ones_v.at[pl.ds(0, n)],
                        cacc.at[pl.ds(s * ROWS_PT + r * CHUNK, n)])
            iones = jnp.ones((L,), jnp.int32)

            def corow(r, _):
                ones_v[r, pl.ds(0, L)] = iones
                return 0

            lax.fori_loop(0, CHUNK, corow, 0)
        plsc.subcore_barrier()

        def gather(j, b):
            pltpu.async_copy(h_hbm.at[src_v.at[j]], bufs[b], gsem[b])

        def wait_gather(j, b):
            pltpu.make_async_copy(h_hbm.at[src_v.at[j]], bufs[b],
                                  gsem[b]).wait()

        def scatter(j, b):
            pltpu.async_copy(bufs[b], acc.at[dst_v.at[j]], ssem[b], add=True)

        def wait_scatter(j, b):
            pltpu.make_async_copy(bufs[b], acc.at[dst_v.at[j]],
                                  ssem[b]).wait()

        for j in range(PF):
            gather(j, j % NB)

        NCNT = CPT_CNT // (CPT // NB)

        def body(t, _):
            for b in range(NB):
                j = t * NB + b
                wait_gather(j, b)
                scatter(j, b)
                bq = (b + PF) % NB
                k = j + PF - NB

                @pl.when(k >= 0)
                def _():
                    wait_scatter(k, bq)

                @pl.when(j + PF < CPT)
                def _():
                    gather(j + PF, bq)
            if with_count:
                for q in range(NCNT):
                    jq = c * CPT_CNT + t * NCNT + q
                    pltpu.sync_copy(ones_v, cacc.at[dst_v.at[jq]], add=True)
            return 0

        lax.fori_loop(0, CPT // NB, body, 0)
        for j in range(CPT + PF - NB, CPT):
            wait_scatter(j, j % NB)
        plsc.subcore_barrier()

        pltpu.sync_copy(acc.at[pl.ds(s * ROWS_PT, ROWS_PT)],
                        out_hbm.at[c, pl.ds(s * ROWS_PT, ROWS_PT)])
        if with_count:
            pltpu.sync_copy(cacc.at[pl.ds(s * ROWS_PT, ROWS_PT)],
                            cnt_hbm.at[c, pl.ds(s * ROWS_PT, ROWS_PT)])

    return sc_agg


_sc_aggs = {}


def _get_sc_agg(with_count):
    if with_count not in _sc_aggs:
        _sc_aggs[with_count] = _make_sc_agg(with_count)
    return _sc_aggs[with_count]


def _embed_body(x_ref, w_ref, b_ref, o_ref, ob_ref):
    h = (jnp.dot(x_ref[...], w_ref[...], preferred_element_type=jnp.float32)
         + b_ref[...])
    o_ref[...] = h
    ob_ref[...] = h.astype(jnp.bfloat16)


def _tc_embed(x, w, b2d):
    return pl.pallas_call(
        _embed_body,
        out_shape=[jax.ShapeDtypeStruct((N_NODES, D), jnp.float32),
                   jax.ShapeDtypeStruct((N_NODES, D), jnp.bfloat16)],
        grid=(N_NODES // BM,),
        in_specs=[
            pl.BlockSpec((BM, D), lambda i: (i, 0)),
            pl.BlockSpec((D, D), lambda i: (0, 0)),
            pl.BlockSpec((1, D), lambda i: (0, 0)),
        ],
        out_specs=[pl.BlockSpec((BM, D), lambda i: (i, 0)),
                   pl.BlockSpec((BM, D), lambda i: (i, 0))],
    )(x, w, b2d)


def _mfconv_core(p0_ref, p1_ref, x_ref, c0_ref, c1_ref,
                 wl_ref, wr_ref, bs_ref, relu):
    agg = jnp.concatenate([p0_ref[...], p1_ref[...]],
                          axis=1).astype(jnp.float32)
    x = x_ref[...]
    deg = jnp.minimum(c0_ref[...][:, 0:1] + c1_ref[...][:, 0:1], MAX_DEG)
    iot = lax.broadcasted_iota(jnp.int32, (BM, MAX_DEG + 1), 1)
    onehot = deg == iot
    out = jnp.dot(onehot.astype(jnp.float32), bs_ref[...],
                  preferred_element_type=jnp.float32)
    for i in range(MAX_DEG + 1):
        t = (jnp.dot(agg, wl_ref[i], preferred_element_type=jnp.float32)
             + jnp.dot(x, wr_ref[i], preferred_element_type=jnp.float32))
        out = out + jnp.where(onehot[:, i:i + 1], t, 0.0)
    if relu:
        out = jnp.maximum(out, 0.0)
    return out


def _mfconv_body(p0_ref, p1_ref, x_ref, c0_ref, c1_ref,
                 wl_ref, wr_ref, bs_ref, o_ref, ob_ref):
    out = _mfconv_core(p0_ref, p1_ref, x_ref, c0_ref, c1_ref,
                       wl_ref, wr_ref, bs_ref, relu=True)
    o_ref[...] = out
    ob_ref[...] = out.astype(jnp.bfloat16)


_MF_IN_SPECS = [
    pl.BlockSpec((BM, HD), lambda i: (i, 0)),
    pl.BlockSpec((BM, HD), lambda i: (i, 0)),
    pl.BlockSpec((BM, D), lambda i: (i, 0)),
    pl.BlockSpec((BM, L), lambda i: (i, 0)),
    pl.BlockSpec((BM, L), lambda i: (i, 0)),
    pl.BlockSpec((MAX_DEG + 1, D, D), lambda i: (0, 0, 0)),
    pl.BlockSpec((MAX_DEG + 1, D, D), lambda i: (0, 0, 0)),
    pl.BlockSpec((MAX_DEG + 1, D), lambda i: (0, 0)),
]


def _tc_mfconv(p0, p1, x, c0, c1, wl, wr, bsum):
    return pl.pallas_call(
        _mfconv_body,
        out_shape=[jax.ShapeDtypeStruct((N_NODES, D), jnp.float32),
                   jax.ShapeDtypeStruct((N_NODES, D), jnp.bfloat16)],
        grid=(N_NODES // BM,),
        in_specs=_MF_IN_SPECS,
        out_specs=[pl.BlockSpec((BM, D), lambda i: (i, 0)),
                   pl.BlockSpec((BM, D), lambda i: (i, 0))],
    )(p0, p1, x, c0, c1, wl, wr, bsum)


def _mfconv_pool_body(p0_ref, p1_ref, x_ref, c0_ref, c1_ref,
                      wl_ref, wr_ref, bs_ref, b_ref,
                      w1_ref, b1_ref, w2_ref, b2_ref, o_ref,
                      sums, counts):
    i = pl.program_id(0)
    nsteps = pl.num_programs(0)

    @pl.when(i == 0)
    def _():
        sums[...] = jnp.zeros_like(sums)
        counts[...] = jnp.zeros_like(counts)

    h = _mfconv_core(p0_ref, p1_ref, x_ref, c0_ref, c1_ref,
                     wl_ref, wr_ref, bs_ref, relu=False)
    bi = b_ref[...][:, 0:1]
    iot = lax.broadcasted_iota(jnp.int32, (BM, N_GRAPHS), 1)
    onehot = (bi == iot).astype(jnp.float32)
    sums[...] += jax.lax.dot_general(
        onehot, h, (((0,), (0,)), ((), ())),
        preferred_element_type=jnp.float32)
    counts[...] += jax.lax.dot_general(
        onehot, jnp.ones((BM, D), jnp.float32), (((0,), (0,)), ((), ())),
        preferred_element_type=jnp.float32)

    @pl.when(i == nsteps - 1)
    def _():
        pooled = sums[...] / jnp.maximum(counts[...], 1.0)
        z = jnp.maximum(
            jnp.dot(pooled, w1_ref[...], preferred_element_type=jnp.float32)
            + b1_ref[...], 0.0)
        o_ref[...] = (jnp.dot(z, w2_ref[...],
                              preferred_element_type=jnp.float32)
                      + b2_ref[...])


def _tc_mfconv_pool(p0, p1, x, c0, c1, wl, wr, bsum,
                    batch2d, w1, b1_2d, w2_pad, b2_2d):
    return pl.pallas_call(
        _mfconv_pool_body,
        out_shape=jax.ShapeDtypeStruct((N_GRAPHS, D), jnp.float32),
        grid=(N_NODES // BM,),
        in_specs=_MF_IN_SPECS + [
            pl.BlockSpec((BM, 1), lambda i: (i, 0)),
            pl.BlockSpec((D, D), lambda i: (0, 0)),
            pl.BlockSpec((1, D), lambda i: (0, 0)),
            pl.BlockSpec((D, D), lambda i: (0, 0)),
            pl.BlockSpec((1, D), lambda i: (0, 0)),
        ],
        out_specs=pl.BlockSpec((N_GRAPHS, D), lambda i: (0, 0)),
        scratch_shapes=[
            pltpu.VMEM((N_GRAPHS, D), jnp.float32),
            pltpu.VMEM((N_GRAPHS, D), jnp.float32),
        ],
    )(p0, p1, x, c0, c1, wl, wr, bsum,
      batch2d, w1, b1_2d, w2_pad, b2_2d)


def kernel(x, edge_index, edge_attr, batch_idx, embed_W, embed_b,
           Wl1, bl1, Wr1, br1, Wl2, bl2, Wr2, br2,
           lin1_W, lin1_b, lin2_W, lin2_b):
    del edge_attr

    src = edge_index[0].astype(jnp.int32)
    dst = edge_index[1].astype(jnp.int32)
    pad = EDGES_PAD - N_EDGES
    src2 = jnp.pad(2 * src, (0, pad)).reshape(NS, CPT, CHUNK)
    srcs2 = jnp.stack([src2, src2 + 1])
    dsts = jnp.pad(dst, (0, pad), constant_values=N_NODES).reshape(
        NS, CPT, CHUNK)

    h0, h0_bf = _tc_embed(x, embed_W, embed_b.reshape(1, D))

    p, cnt = _get_sc_agg(True)(h0_bf.reshape(2 * N_NODES, HD), srcs2, dsts)
    c0 = cnt[0, :N_NODES]
    c1 = cnt[1, :N_NODES]
    g1, g1_bf = _tc_mfconv(p[0, :N_NODES], p[1, :N_NODES], h0, c0, c1,
                           Wl1, Wr1, bl1 + br1)

    (p2,) = _get_sc_agg(False)(g1_bf.reshape(2 * N_NODES, HD), srcs2, dsts)
    w2_pad = jnp.pad(lin2_W, ((0, 0), (0, D - 1)))
    b2_2d = jnp.pad(lin2_b.reshape(1, 1), ((0, 0), (0, D - 1)))
    res = _tc_mfconv_pool(p2[0, :N_NODES], p2[1, :N_NODES], g1, c0, c1,
                          Wl2, Wr2, bl2 + br2,
                          batch_idx.astype(jnp.int32).reshape(N_NODES, 1),
                          lin1_W, lin1_b.reshape(1, D), w2_pad, b2_2d)
    return res[:, 0:1]


# speedup vs baseline: 1.0049x; 1.0049x over previous
import functools

import jax
import jax.numpy as jnp
from jax import lax
from jax.experimental import pallas as pl
from jax.experimental.pallas import tpu as pltpu
from jax.experimental.pallas import tpu_sc as plsc

N_NODES = 10000
D = 128
N_EDGES = 320000
MAX_DEG = 10
N_GRAPHS = 64

NC, NS, L = 2, 16, 16
NW = NC * NS
CHUNK = 128
CPT = 160
EDGES_PAD = NS * CPT * CHUNK
HD = D // NC
NPAD = 10016
ROWS_PT = NPAD // NS
CPT_CNT = CPT // NC

BM = 2000


NB = 8
PF = 3


def _make_sc_agg(with_count):
    mesh = plsc.VectorSubcoreMesh(
        core_axis_name="c", subcore_axis_name="s",
        num_cores=NC, num_subcores=NS)

    out_type = [jax.ShapeDtypeStruct((NC, NPAD, HD), jnp.bfloat16)]
    scratch = [
        pltpu.VMEM((CPT, CHUNK), jnp.int32),
        pltpu.VMEM((CPT, CHUNK), jnp.int32),
        [pltpu.VMEM((CHUNK, HD), jnp.bfloat16)] * NB,
        pltpu.VMEM_SHARED((NPAD, HD), jnp.bfloat16),
        [pltpu.SemaphoreType.DMA] * NB,
        [pltpu.SemaphoreType.DMA] * NB,
    ]
    if with_count:
        out_type.append(jax.ShapeDtypeStruct((NC, NPAD, L), jnp.int32))
        scratch.append(pltpu.VMEM((CHUNK, L), jnp.int32))
        scratch.append(pltpu.VMEM_SHARED((NPAD, L), jnp.int32))
        scratch.append(pltpu.SemaphoreType.DMA)

    @functools.partial(
        pl.kernel,
        out_type=out_type,
        mesh=mesh,
        compiler_params=pltpu.CompilerParams(use_tc_tiling_on_sc=False),
        scratch_types=scratch,
    )
    def sc_agg(h_hbm, srcs_hbm, dsts_hbm, *refs):
        if with_count:
            (out_hbm, cnt_hbm, src_v, dst_v, bufs, acc, gsem, ssem,
             ones_v, cacc, csem) = refs
        else:
            out_hbm, src_v, dst_v, bufs, acc, gsem, ssem = refs
        c = lax.axis_index("c")
        s = lax.axis_index("s")

        pltpu.sync_copy(srcs_hbm.at[c, s], src_v)
        pltpu.sync_copy(dsts_hbm.at[s], dst_v)

        zeros = jnp.zeros((2 * L,), jnp.bfloat16)

        def zrow(r, _):
            for cc in range(HD // (2 * L)):
                bufs[0][r, pl.ds(cc * 2 * L, 2 * L)] = zeros
            return 0

        lax.fori_loop(0, CHUNK, zrow, 0)
        for r in range(ROWS_PT // CHUNK + 1):
            n = min(CHUNK, ROWS_PT - r * CHUNK)
            if n > 0:
                pltpu.sync_copy(
                    bufs[0].at[pl.ds(0, n)],
                    acc.at[pl.ds(s * ROWS_PT + r * CHUNK, n)])

        if with_count:
            izeros = jnp.zeros((L,), jnp.int32)

            def czrow(r, _):
                ones_v[r, pl.ds(0, L)] = izeros
                return 0

            lax.fori_loop(0, CHUNK, czrow, 0)
            for r in range(ROWS_PT // CHUNK + 1):
                n = min(CHUNK, ROWS_PT - r * CHUNK)
                if n > 0:
                    pltpu.sync_copy(
                        ones_v.at[pl.ds(0, n)],
                        cacc.at[pl.ds(s * ROWS_PT + r * CHUNK, n)])
            iones = jnp.ones((L,), jnp.int32)

            def corow(r, _):
                ones_v[r, pl.ds(0, L)] = iones
                return 0

            lax.fori_loop(0, CHUNK, corow, 0)
        plsc.subcore_barrier()

        def gather(j, b):
            pltpu.async_copy(h_hbm.at[src_v.at[j]], bufs[b], gsem[b])

        def wait_gather(j, b):
            pltpu.make_async_copy(h_hbm.at[src_v.at[j]], bufs[b],
                                  gsem[b]).wait()

        def scatter(j, b):
            pltpu.async_copy(bufs[b], acc.at[dst_v.at[j]], ssem[b], add=True)

        def wait_scatter(j, b):
            pltpu.make_async_copy(bufs[b], acc.at[dst_v.at[j]],
                                  ssem[b]).wait()

        for j in range(PF):
            gather(j, j % NB)

        NCNT = CPT_CNT // (CPT // NB)

        def body(t, _):
            for b in range(NB):
                j = t * NB + b
                wait_gather(j, b)
                scatter(j, b)
                bq = (b + PF) % NB
                k = j + PF - NB

                @pl.when(k >= 0)
                def _():
                    wait_scatter(k, bq)

                @pl.when(j + PF < CPT)
                def _():
                    gather(j + PF, bq)
            if with_count:
                for q in range(NCNT):
                    jq = c * CPT_CNT + t * NCNT + q
                    pltpu.async_copy(ones_v, cacc.at[dst_v.at[jq]], csem,
                                     add=True)
            return 0

        lax.fori_loop(0, CPT // NB, body, 0)
        for j in range(CPT + PF - NB, CPT):
            wait_scatter(j, j % NB)
        if with_count:
            def cdrain(j, _):
                pltpu.make_async_copy(ones_v, cacc.at[dst_v.at[j]],
                                      csem).wait()
                return 0
            lax.fori_loop(0, CPT_CNT, cdrain, 0)
        plsc.subcore_barrier()

        pltpu.sync_copy(acc.at[pl.ds(s * ROWS_PT, ROWS_PT)],
                        out_hbm.at[c, pl.ds(s * ROWS_PT, ROWS_PT)])
        if with_count:
            pltpu.sync_copy(cacc.at[pl.ds(s * ROWS_PT, ROWS_PT)],
                            cnt_hbm.at[c, pl.ds(s * ROWS_PT, ROWS_PT)])

    return sc_agg


_sc_aggs = {}


def _get_sc_agg(with_count):
    if with_count not in _sc_aggs:
        _sc_aggs[with_count] = _make_sc_agg(with_count)
    return _sc_aggs[with_count]


def _embed_body(x_ref, w_ref, b_ref, o_ref, ob_ref):
    h = (jnp.dot(x_ref[...], w_ref[...], preferred_element_type=jnp.float32)
         + b_ref[...])
    o_ref[...] = h
    ob_ref[...] = h.astype(jnp.bfloat16)


def _tc_embed(x, w, b2d):
    return pl.pallas_call(
        _embed_body,
        out_shape=[jax.ShapeDtypeStruct((N_NODES, D), jnp.float32),
                   jax.ShapeDtypeStruct((N_NODES, D), jnp.bfloat16)],
        grid=(N_NODES // BM,),
        in_specs=[
            pl.BlockSpec((BM, D), lambda i: (i, 0)),
            pl.BlockSpec((D, D), lambda i: (0, 0)),
            pl.BlockSpec((1, D), lambda i: (0, 0)),
        ],
        out_specs=[pl.BlockSpec((BM, D), lambda i: (i, 0)),
                   pl.BlockSpec((BM, D), lambda i: (i, 0))],
    )(x, w, b2d)


def _mfconv_core(p0_ref, p1_ref, x_ref, c0_ref, c1_ref,
                 wl_ref, wr_ref, bs_ref, relu):
    agg = jnp.concatenate([p0_ref[...], p1_ref[...]],
                          axis=1).astype(jnp.float32)
    x = x_ref[...]
    deg = jnp.minimum(c0_ref[...][:, 0:1] + c1_ref[...][:, 0:1], MAX_DEG)
    iot = lax.broadcasted_iota(jnp.int32, (BM, MAX_DEG + 1), 1)
    onehot = deg == iot
    out = jnp.dot(onehot.astype(jnp.float32), bs_ref[...],
                  preferred_element_type=jnp.float32)
    for i in range(MAX_DEG + 1):
        t = (jnp.dot(agg, wl_ref[i], preferred_element_type=jnp.float32)
             + jnp.dot(x, wr_ref[i], preferred_element_type=jnp.float32))
        out = out + jnp.where(onehot[:, i:i + 1], t, 0.0)
    if relu:
        out = jnp.maximum(out, 0.0)
    return out


def _mfconv_body(p0_ref, p1_ref, x_ref, c0_ref, c1_ref,
                 wl_ref, wr_ref, bs_ref, o_ref, ob_ref):
    out = _mfconv_core(p0_ref, p1_ref, x_ref, c0_ref, c1_ref,
                       wl_ref, wr_ref, bs_ref, relu=True)
    o_ref[...] = out
    ob_ref[...] = out.astype(jnp.bfloat16)


_MF_IN_SPECS = [
    pl.BlockSpec((BM, HD), lambda i: (i, 0)),
    pl.BlockSpec((BM, HD), lambda i: (i, 0)),
    pl.BlockSpec((BM, D), lambda i: (i, 0)),
    pl.BlockSpec((BM, L), lambda i: (i, 0)),
    pl.BlockSpec((BM, L), lambda i: (i, 0)),
    pl.BlockSpec((MAX_DEG + 1, D, D), lambda i: (0, 0, 0)),
    pl.BlockSpec((MAX_DEG + 1, D, D), lambda i: (0, 0, 0)),
    pl.BlockSpec((MAX_DEG + 1, D), lambda i: (0, 0)),
]


def _tc_mfconv(p0, p1, x, c0, c1, wl, wr, bsum):
    return pl.pallas_call(
        _mfconv_body,
        out_shape=[jax.ShapeDtypeStruct((N_NODES, D), jnp.float32),
                   jax.ShapeDtypeStruct((N_NODES, D), jnp.bfloat16)],
        grid=(N_NODES // BM,),
        in_specs=_MF_IN_SPECS,
        out_specs=[pl.BlockSpec((BM, D), lambda i: (i, 0)),
                   pl.BlockSpec((BM, D), lambda i: (i, 0))],
    )(p0, p1, x, c0, c1, wl, wr, bsum)


def _mfconv_pool_body(p0_ref, p1_ref, x_ref, c0_ref, c1_ref,
                      wl_ref, wr_ref, bs_ref, b_ref,
                      w1_ref, b1_ref, w2_ref, b2_ref, o_ref,
                      sums, counts):
    i = pl.program_id(0)
    nsteps = pl.num_programs(0)

    @pl.when(i == 0)
    def _():
        sums[...] = jnp.zeros_like(sums)
        counts[...] = jnp.zeros_like(counts)

    h = _mfconv_core(p0_ref, p1_ref, x_ref, c0_ref, c1_ref,
                     wl_ref, wr_ref, bs_ref, relu=False)
    bi = b_ref[...][:, 0:1]
    iot = lax.broadcasted_iota(jnp.int32, (BM, N_GRAPHS), 1)
    onehot = (bi == iot).astype(jnp.float32)
    sums[...] += jax.lax.dot_general(
        onehot, h, (((0,), (0,)), ((), ())),
        preferred_element_type=jnp.float32)
    counts[...] += jax.lax.dot_general(
        onehot, jnp.ones((BM, D), jnp.float32), (((0,), (0,)), ((), ())),
        preferred_element_type=jnp.float32)

    @pl.when(i == nsteps - 1)
    def _():
        pooled = sums[...] / jnp.maximum(counts[...], 1.0)
        z = jnp.maximum(
            jnp.dot(pooled, w1_ref[...], preferred_element_type=jnp.float32)
            + b1_ref[...], 0.0)
        o_ref[...] = (jnp.dot(z, w2_ref[...],
                              preferred_element_type=jnp.float32)
                      + b2_ref[...])


def _tc_mfconv_pool(p0, p1, x, c0, c1, wl, wr, bsum,
                    batch2d, w1, b1_2d, w2_pad, b2_2d):
    return pl.pallas_call(
        _mfconv_pool_body,
        out_shape=jax.ShapeDtypeStruct((N_GRAPHS, D), jnp.float32),
        grid=(N_NODES // BM,),
        in_specs=_MF_IN_SPECS + [
            pl.BlockSpec((BM, 1), lambda i: (i, 0)),
            pl.BlockSpec((D, D), lambda i: (0, 0)),
            pl.BlockSpec((1, D), lambda i: (0, 0)),
            pl.BlockSpec((D, D), lambda i: (0, 0)),
            pl.BlockSpec((1, D), lambda i: (0, 0)),
        ],
        out_specs=pl.BlockSpec((N_GRAPHS, D), lambda i: (0, 0)),
        scratch_shapes=[
            pltpu.VMEM((N_GRAPHS, D), jnp.float32),
            pltpu.VMEM((N_GRAPHS, D), jnp.float32),
        ],
    )(p0, p1, x, c0, c1, wl, wr, bsum,
      batch2d, w1, b1_2d, w2_pad, b2_2d)


def kernel(x, edge_index, edge_attr, batch_idx, embed_W, embed_b,
           Wl1, bl1, Wr1, br1, Wl2, bl2, Wr2, br2,
           lin1_W, lin1_b, lin2_W, lin2_b):
    del edge_attr

    src = edge_index[0].astype(jnp.int32)
    dst = edge_index[1].astype(jnp.int32)
    pad = EDGES_PAD - N_EDGES
    src2 = jnp.pad(2 * src, (0, pad)).reshape(NS, CPT, CHUNK)
    srcs2 = jnp.stack([src2, src2 + 1])
    dsts = jnp.pad(dst, (0, pad), constant_values=N_NODES).reshape(
        NS, CPT, CHUNK)

    h0, h0_bf = _tc_embed(x, embed_W, embed_b.reshape(1, D))

    p, cnt = _get_sc_agg(True)(h0_bf.reshape(2 * N_NODES, HD), srcs2, dsts)
    c0 = cnt[0, :N_NODES]
    c1 = cnt[1, :N_NODES]
    g1, g1_bf = _tc_mfconv(p[0, :N_NODES], p[1, :N_NODES], h0, c0, c1,
                           Wl1, Wr1, bl1 + br1)

    (p2,) = _get_sc_agg(False)(g1_bf.reshape(2 * N_NODES, HD), srcs2, dsts)
    w2_pad = jnp.pad(lin2_W, ((0, 0), (0, D - 1)))
    b2_2d = jnp.pad(lin2_b.reshape(1, 1), ((0, 0), (0, D - 1)))
    res = _tc_mfconv_pool(p2[0, :N_NODES], p2[1, :N_NODES], g1, c0, c1,
                          Wl2, Wr2, bl2 + br2,
                          batch_idx.astype(jnp.int32).reshape(N_NODES, 1),
                          lin1_W, lin1_b.reshape(1, D), w2_pad, b2_2d)
    return res[:, 0:1]


# speedup vs baseline: 1.0126x; 1.0076x over previous
import functools

import jax
import jax.numpy as jnp
from jax import lax
from jax.experimental import pallas as pl
from jax.experimental.pallas import tpu as pltpu
from jax.experimental.pallas import tpu_sc as plsc

N_NODES = 10000
D = 128
N_EDGES = 320000
MAX_DEG = 10
N_GRAPHS = 64

NC, NS, L = 2, 16, 16
NW = NC * NS
CHUNK = 128
CPT = 160
EDGES_PAD = NS * CPT * CHUNK
HD = D // NC
NPAD = 10016
ROWS_PT = NPAD // NS
CPT_CNT = CPT // NC

BM = 2000


NB = 8
PF = 3


def _make_sc_agg(with_count):
    mesh = plsc.VectorSubcoreMesh(
        core_axis_name="c", subcore_axis_name="s",
        num_cores=NC, num_subcores=NS)

    out_type = [jax.ShapeDtypeStruct((NC, NPAD, HD), jnp.bfloat16)]
    scratch = [
        pltpu.VMEM((CPT, CHUNK), jnp.int32),
        pltpu.VMEM((CPT, CHUNK), jnp.int32),
        [pltpu.VMEM((CHUNK, HD), jnp.bfloat16)] * NB,
        pltpu.VMEM_SHARED((NPAD, HD), jnp.bfloat16),
        [pltpu.SemaphoreType.DMA] * NB,
        [pltpu.SemaphoreType.DMA] * NB,
    ]
    if with_count:
        out_type.append(jax.ShapeDtypeStruct((NC, NPAD, L), jnp.int32))
        scratch.append(pltpu.VMEM((CHUNK, L), jnp.int32))
        scratch.append(pltpu.VMEM_SHARED((NPAD, L), jnp.int32))
        scratch.append(pltpu.SemaphoreType.DMA)

    @functools.partial(
        pl.kernel,
        out_type=out_type,
        mesh=mesh,
        compiler_params=pltpu.CompilerParams(use_tc_tiling_on_sc=False),
        scratch_types=scratch,
    )
    def sc_agg(h_hbm, srcs_hbm, dsts_hbm, *refs):
        if with_count:
            (out_hbm, cnt_hbm, src_v, dst_v, bufs, acc, gsem, ssem,
             ones_v, cacc, csem) = refs
        else:
            out_hbm, src_v, dst_v, bufs, acc, gsem, ssem = refs
        c = lax.axis_index("c")
        s = lax.axis_index("s")

        pltpu.sync_copy(srcs_hbm.at[c, s], src_v)
        pltpu.sync_copy(dsts_hbm.at[s], dst_v)

        zeros = jnp.zeros((2 * L,), jnp.bfloat16)

        def zrow(r, _):
            for cc in range(HD // (2 * L)):
                bufs[0][r, pl.ds(cc * 2 * L, 2 * L)] = zeros
            return 0

        lax.fori_loop(0, CHUNK, zrow, 0)
        for r in range(ROWS_PT // CHUNK + 1):
            n = min(CHUNK, ROWS_PT - r * CHUNK)
            if n > 0:
                pltpu.sync_copy(
                    bufs[0].at[pl.ds(0, n)],
                    acc.at[pl.ds(s * ROWS_PT + r * CHUNK, n)])

        if with_count:
            izeros = jnp.zeros((L,), jnp.int32)

            def czrow(r, _):
                ones_v[r, pl.ds(0, L)] = izeros
                return 0

            lax.fori_loop(0, CHUNK, czrow, 0)
            for r in range(ROWS_PT // CHUNK + 1):
                n = min(CHUNK, ROWS_PT - r * CHUNK)
                if n > 0:
                    pltpu.sync_copy(
                        ones_v.at[pl.ds(0, n)],
                        cacc.at[pl.ds(s * ROWS_PT + r * CHUNK, n)])
            iones = jnp.ones((L,), jnp.int32)

            def corow(r, _):
                ones_v[r, pl.ds(0, L)] = iones
                return 0

            lax.fori_loop(0, CHUNK, corow, 0)
        plsc.subcore_barrier()

        def gather(j, b):
            pltpu.async_copy(h_hbm.at[src_v.at[j]], bufs[b], gsem[b])

        def wait_gather(j, b):
            pltpu.make_async_copy(h_hbm.at[src_v.at[j]], bufs[b],
                                  gsem[b]).wait()

        def scatter(j, b):
            pltpu.async_copy(bufs[b], acc.at[dst_v.at[j]], ssem[b], add=True)

        def wait_scatter(j, b):
            pltpu.make_async_copy(bufs[b], acc.at[dst_v.at[j]],
                                  ssem[b]).wait()

        for j in range(PF):
            gather(j, j % NB)

        NCNT = CPT_CNT // (CPT // NB)

        def body(t, _):
            for b in range(NB):
                j = t * NB + b
                wait_gather(j, b)
                scatter(j, b)
                bq = (b + PF) % NB
                k = j + PF - NB

                @pl.when(k >= 0)
                def _():
                    wait_scatter(k, bq)

                @pl.when(j + PF < CPT)
                def _():
                    gather(j + PF, bq)
            if with_count:
                for q in range(NCNT):
                    jq = c * CPT_CNT + t * NCNT + q
                    pltpu.async_copy(ones_v, cacc.at[dst_v.at[jq]], csem,
                                     add=True)
            return 0

        lax.fori_loop(0, CPT // NB, body, 0)
        for j in range(CPT + PF - NB, CPT):
            wait_scatter(j, j % NB)
        if with_count:
            def cdrain(j, _):
                pltpu.make_async_copy(ones_v, cacc.at[dst_v.at[j]],
                                      csem).wait()
                return 0
            lax.fori_loop(0, CPT_CNT, cdrain, 0)
        plsc.subcore_barrier()

        pltpu.sync_copy(acc.at[pl.ds(s * ROWS_PT, ROWS_PT)],
                        out_hbm.at[c, pl.ds(s * ROWS_PT, ROWS_PT)])
        if with_count:
            pltpu.sync_copy(cacc.at[pl.ds(s * ROWS_PT, ROWS_PT)],
                            cnt_hbm.at[c, pl.ds(s * ROWS_PT, ROWS_PT)])

    return sc_agg


_sc_aggs = {}


def _get_sc_agg(with_count):
    if with_count not in _sc_aggs:
        _sc_aggs[with_count] = _make_sc_agg(with_count)
    return _sc_aggs[with_count]


def _make_sc_cnt():
    mesh = plsc.VectorSubcoreMesh(
        core_axis_name="c", subcore_axis_name="s",
        num_cores=NC, num_subcores=NS)

    @functools.partial(
        pl.kernel,
        out_type=jax.ShapeDtypeStruct((NC, NPAD, L), jnp.int32),
        mesh=mesh,
        compiler_params=pltpu.CompilerParams(use_tc_tiling_on_sc=False),
        scratch_types=[
            pltpu.VMEM((CPT_CNT, CHUNK), jnp.int32),
            pltpu.VMEM((CHUNK, L), jnp.int32),
            pltpu.VMEM((CHUNK, L), jnp.int32),
            pltpu.VMEM_SHARED((NPAD, L), jnp.int32),
        ],
    )
    def sc_cnt(dsts_hbm, out_hbm, dst_v, ones_v, zbuf, acc):
        c = lax.axis_index("c")
        s = lax.axis_index("s")

        pltpu.sync_copy(dsts_hbm.at[s, pl.ds(c * CPT_CNT, CPT_CNT)], dst_v)

        ones = jnp.ones((L,), jnp.int32)
        zeros = jnp.zeros((L,), jnp.int32)

        def fill(r, _):
            ones_v[r, pl.ds(0, L)] = ones
            zbuf[r, pl.ds(0, L)] = zeros
            return 0

        lax.fori_loop(0, CHUNK, fill, 0)
        for r in range(ROWS_PT // CHUNK + 1):
            n = min(CHUNK, ROWS_PT - r * CHUNK)
            if n > 0:
                pltpu.sync_copy(
                    zbuf.at[pl.ds(0, n)],
                    acc.at[pl.ds(s * ROWS_PT + r * CHUNK, n)])
        plsc.subcore_barrier()

        def body(j, _):
            pltpu.sync_copy(ones_v, acc.at[dst_v.at[j]], add=True)
            return 0

        lax.fori_loop(0, CPT_CNT, body, 0)
        plsc.subcore_barrier()

        pltpu.sync_copy(acc.at[pl.ds(s * ROWS_PT, ROWS_PT)],
                        out_hbm.at[c, pl.ds(s * ROWS_PT, ROWS_PT)])

    return sc_cnt


_sc_cnt = None


def _get_sc_cnt():
    global _sc_cnt
    if _sc_cnt is None:
        _sc_cnt = _make_sc_cnt()
    return _sc_cnt


def _embed_body(x_ref, w_ref, b_ref, o_ref, ob_ref):
    h = (jnp.dot(x_ref[...], w_ref[...], preferred_element_type=jnp.float32)
         + b_ref[...])
    o_ref[...] = h
    ob_ref[...] = h.astype(jnp.bfloat16)


def _tc_embed(x, w, b2d):
    return pl.pallas_call(
        _embed_body,
        out_shape=[jax.ShapeDtypeStruct((N_NODES, D), jnp.float32),
                   jax.ShapeDtypeStruct((N_NODES, D), jnp.bfloat16)],
        grid=(N_NODES // BM,),
        in_specs=[
            pl.BlockSpec((BM, D), lambda i: (i, 0)),
            pl.BlockSpec((D, D), lambda i: (0, 0)),
            pl.BlockSpec((1, D), lambda i: (0, 0)),
        ],
        out_specs=[pl.BlockSpec((BM, D), lambda i: (i, 0)),
                   pl.BlockSpec((BM, D), lambda i: (i, 0))],
    )(x, w, b2d)


def _mfconv_core(p0_ref, p1_ref, x_ref, c0_ref, c1_ref,
                 wl_ref, wr_ref, bs_ref, relu):
    agg = jnp.concatenate([p0_ref[...], p1_ref[...]],
                          axis=1).astype(jnp.float32)
    x = x_ref[...]
    deg = jnp.minimum(c0_ref[...][:, 0:1] + c1_ref[...][:, 0:1], MAX_DEG)
    iot = lax.broadcasted_iota(jnp.int32, (BM, MAX_DEG + 1), 1)
    onehot = deg == iot
    out = jnp.dot(onehot.astype(jnp.float32), bs_ref[...],
                  preferred_element_type=jnp.float32)
    for i in range(MAX_DEG + 1):
        t = (jnp.dot(agg, wl_ref[i], preferred_element_type=jnp.float32)
             + jnp.dot(x, wr_ref[i], preferred_element_type=jnp.float32))
        out = out + jnp.where(onehot[:, i:i + 1], t, 0.0)
    if relu:
        out = jnp.maximum(out, 0.0)
    return out


def _mfconv_body(p0_ref, p1_ref, x_ref, c0_ref, c1_ref,
                 wl_ref, wr_ref, bs_ref, o_ref, ob_ref):
    out = _mfconv_core(p0_ref, p1_ref, x_ref, c0_ref, c1_ref,
                       wl_ref, wr_ref, bs_ref, relu=True)
    o_ref[...] = out
    ob_ref[...] = out.astype(jnp.bfloat16)


_MF_IN_SPECS = [
    pl.BlockSpec((BM, HD), lambda i: (i, 0)),
    pl.BlockSpec((BM, HD), lambda i: (i, 0)),
    pl.BlockSpec((BM, D), lambda i: (i, 0)),
    pl.BlockSpec((BM, L), lambda i: (i, 0)),
    pl.BlockSpec((BM, L), lambda i: (i, 0)),
    pl.BlockSpec((MAX_DEG + 1, D, D), lambda i: (0, 0, 0)),
    pl.BlockSpec((MAX_DEG + 1, D, D), lambda i: (0, 0, 0)),
    pl.BlockSpec((MAX_DEG + 1, D), lambda i: (0, 0)),
]


def _tc_mfconv(p0, p1, x, c0, c1, wl, wr, bsum):
    return pl.pallas_call(
        _mfconv_body,
        out_shape=[jax.ShapeDtypeStruct((N_NODES, D), jnp.float32),
                   jax.ShapeDtypeStruct((N_NODES, D), jnp.bfloat16)],
        grid=(N_NODES // BM,),
        in_specs=_MF_IN_SPECS,
        out_specs=[pl.BlockSpec((BM, D), lambda i: (i, 0)),
                   pl.BlockSpec((BM, D), lambda i: (i, 0))],
    )(p0, p1, x, c0, c1, wl, wr, bsum)


def _mfconv_pool_body(p0_ref, p1_ref, x_ref, c0_ref, c1_ref,
                      wl_ref, wr_ref, bs_ref, b_ref,
                      w1_ref, b1_ref, w2_ref, b2_ref, o_ref,
                      sums, counts):
    i = pl.program_id(0)
    nsteps = pl.num_programs(0)

    @pl.when(i == 0)
    def _():
        sums[...] = jnp.zeros_like(sums)
        counts[...] = jnp.zeros_like(counts)

    h = _mfconv_core(p0_ref, p1_ref, x_ref, c0_ref, c1_ref,
                     wl_ref, wr_ref, bs_ref, relu=False)
    bi = b_ref[...][:, 0:1]
    iot = lax.broadcasted_iota(jnp.int32, (BM, N_GRAPHS), 1)
    onehot = (bi == iot).astype(jnp.float32)
    sums[...] += jax.lax.dot_general(
        onehot, h, (((0,), (0,)), ((), ())),
        preferred_element_type=jnp.float32)
    counts[...] += jax.lax.dot_general(
        onehot, jnp.ones((BM, D), jnp.float32), (((0,), (0,)), ((), ())),
        preferred_element_type=jnp.float32)

    @pl.when(i == nsteps - 1)
    def _():
        pooled = sums[...] / jnp.maximum(counts[...], 1.0)
        z = jnp.maximum(
            jnp.dot(pooled, w1_ref[...], preferred_element_type=jnp.float32)
            + b1_ref[...], 0.0)
        o_ref[...] = (jnp.dot(z, w2_ref[...],
                              preferred_element_type=jnp.float32)
                      + b2_ref[...])


def _tc_mfconv_pool(p0, p1, x, c0, c1, wl, wr, bsum,
                    batch2d, w1, b1_2d, w2_pad, b2_2d):
    return pl.pallas_call(
        _mfconv_pool_body,
        out_shape=jax.ShapeDtypeStruct((N_GRAPHS, D), jnp.float32),
        grid=(N_NODES // BM,),
        in_specs=_MF_IN_SPECS + [
            pl.BlockSpec((BM, 1), lambda i: (i, 0)),
            pl.BlockSpec((D, D), lambda i: (0, 0)),
            pl.BlockSpec((1, D), lambda i: (0, 0)),
            pl.BlockSpec((D, D), lambda i: (0, 0)),
            pl.BlockSpec((1, D), lambda i: (0, 0)),
        ],
        out_specs=pl.BlockSpec((N_GRAPHS, D), lambda i: (0, 0)),
        scratch_shapes=[
            pltpu.VMEM((N_GRAPHS, D), jnp.float32),
            pltpu.VMEM((N_GRAPHS, D), jnp.float32),
        ],
    )(p0, p1, x, c0, c1, wl, wr, bsum,
      batch2d, w1, b1_2d, w2_pad, b2_2d)


def kernel(x, edge_index, edge_attr, batch_idx, embed_W, embed_b,
           Wl1, bl1, Wr1, br1, Wl2, bl2, Wr2, br2,
           lin1_W, lin1_b, lin2_W, lin2_b):
    del edge_attr

    src = edge_index[0].astype(jnp.int32)
    dst = edge_index[1].astype(jnp.int32)
    pad = EDGES_PAD - N_EDGES
    src2 = jnp.pad(2 * src, (0, pad)).reshape(NS, CPT, CHUNK)
    srcs2 = jnp.stack([src2, src2 + 1])
    dsts = jnp.pad(dst, (0, pad), constant_values=N_NODES).reshape(
        NS, CPT, CHUNK)

    cnt = _get_sc_cnt()(dsts)
    c0 = cnt[0, :N_NODES]
    c1 = cnt[1, :N_NODES]

    h0, h0_bf = _tc_embed(x, embed_W, embed_b.reshape(1, D))

    (p,) = _get_sc_agg(False)(h0_bf.reshape(2 * N_NODES, HD), srcs2, dsts)
    g1, g1_bf = _tc_mfconv(p[0, :N_NODES], p[1, :N_NODES], h0, c0, c1,
                           Wl1, Wr1, bl1 + br1)

    (p2,) = _get_sc_agg(False)(g1_bf.reshape(2 * N_NODES, HD), srcs2, dsts)
    w2_pad = jnp.pad(lin2_W, ((0, 0), (0, D - 1)))
    b2_2d = jnp.pad(lin2_b.reshape(1, 1), ((0, 0), (0, D - 1)))
    res = _tc_mfconv_pool(p2[0, :N_NODES], p2[1, :N_NODES], g1, c0, c1,
                          Wl2, Wr2, bl2 + br2,
                          batch_idx.astype(jnp.int32).reshape(N_NODES, 1),
                          lin1_W, lin1_b.reshape(1, D), w2_pad, b2_2d)
    return res[:, 0:1]


# speedup vs baseline: 1.0250x; 1.0123x over previous
import functools

import jax
import jax.numpy as jnp
from jax import lax
from jax.experimental import pallas as pl
from jax.experimental.pallas import tpu as pltpu
from jax.experimental.pallas import tpu_sc as plsc

N_NODES = 10000
D = 128
N_EDGES = 320000
MAX_DEG = 10
N_GRAPHS = 64

NC, NS, L = 2, 16, 16
NW = NC * NS
CHUNK = 128
CPT = 160
EDGES_PAD = NS * CPT * CHUNK
HD = D // NC
NPAD = 10016
ROWS_PT = NPAD // NS
CPT_CNT = CPT // NC

BM = 2000


NB = 10
PF = 4


def _make_sc_agg(with_count):
    mesh = plsc.VectorSubcoreMesh(
        core_axis_name="c", subcore_axis_name="s",
        num_cores=NC, num_subcores=NS)

    out_type = [jax.ShapeDtypeStruct((NC, NPAD, HD), jnp.bfloat16)]
    scratch = [
        pltpu.VMEM((CPT, CHUNK), jnp.int32),
        pltpu.VMEM((CPT, CHUNK), jnp.int32),
        [pltpu.VMEM((CHUNK, HD), jnp.bfloat16)] * NB,
        pltpu.VMEM_SHARED((NPAD, HD), jnp.bfloat16),
        [pltpu.SemaphoreType.DMA] * NB,
        [pltpu.SemaphoreType.DMA] * NB,
    ]
    if with_count:
        out_type.append(jax.ShapeDtypeStruct((NC, NPAD, L), jnp.int32))
        scratch.append(pltpu.VMEM((CHUNK, L), jnp.int32))
        scratch.append(pltpu.VMEM_SHARED((NPAD, L), jnp.int32))
        scratch.append(pltpu.SemaphoreType.DMA)

    @functools.partial(
        pl.kernel,
        out_type=out_type,
        mesh=mesh,
        compiler_params=pltpu.CompilerParams(use_tc_tiling_on_sc=False),
        scratch_types=scratch,
    )
    def sc_agg(h_hbm, srcs_hbm, dsts_hbm, *refs):
        if with_count:
            (out_hbm, cnt_hbm, src_v, dst_v, bufs, acc, gsem, ssem,
             ones_v, cacc, csem) = refs
        else:
            out_hbm, src_v, dst_v, bufs, acc, gsem, ssem = refs
        c = lax.axis_index("c")
        s = lax.axis_index("s")

        pltpu.sync_copy(srcs_hbm.at[c, s], src_v)
        pltpu.sync_copy(dsts_hbm.at[s], dst_v)

        zeros = jnp.zeros((2 * L,), jnp.bfloat16)

        def zrow(r, _):
            for cc in range(HD // (2 * L)):
                bufs[0][r, pl.ds(cc * 2 * L, 2 * L)] = zeros
            return 0

        lax.fori_loop(0, CHUNK, zrow, 0)
        for r in range(ROWS_PT // CHUNK + 1):
            n = min(CHUNK, ROWS_PT - r * CHUNK)
            if n > 0:
                pltpu.sync_copy(
                    bufs[0].at[pl.ds(0, n)],
                    acc.at[pl.ds(s * ROWS_PT + r * CHUNK, n)])

        if with_count:
            izeros = jnp.zeros((L,), jnp.int32)

            def czrow(r, _):
                ones_v[r, pl.ds(0, L)] = izeros
                return 0

            lax.fori_loop(0, CHUNK, czrow, 0)
            for r in range(ROWS_PT // CHUNK + 1):
                n = min(CHUNK, ROWS_PT - r * CHUNK)
                if n > 0:
                    pltpu.sync_copy(
                        ones_v.at[pl.ds(0, n)],
                        cacc.at[pl.ds(s * ROWS_PT + r * CHUNK, n)])
            iones = jnp.ones((L,), jnp.int32)

            def corow(r, _):
                ones_v[r, pl.ds(0, L)] = iones
                return 0

            lax.fori_loop(0, CHUNK, corow, 0)
        plsc.subcore_barrier()

        def gather(j, b):
            pltpu.async_copy(h_hbm.at[src_v.at[j]], bufs[b], gsem[b])

        def wait_gather(j, b):
            pltpu.make_async_copy(h_hbm.at[src_v.at[j]], bufs[b],
                                  gsem[b]).wait()

        def scatter(j, b):
            pltpu.async_copy(bufs[b], acc.at[dst_v.at[j]], ssem[b], add=True)

        def wait_scatter(j, b):
            pltpu.make_async_copy(bufs[b], acc.at[dst_v.at[j]],
                                  ssem[b]).wait()

        for j in range(PF):
            gather(j, j % NB)

        NCNT = CPT_CNT // (CPT // NB)

        def body(t, _):
            for b in range(NB):
                j = t * NB + b
                wait_gather(j, b)
                scatter(j, b)
                bq = (b + PF) % NB
                k = j + PF - NB

                @pl.when(k >= 0)
                def _():
                    wait_scatter(k, bq)

                @pl.when(j + PF < CPT)
                def _():
                    gather(j + PF, bq)
            if with_count:
                for q in range(NCNT):
                    jq = c * CPT_CNT + t * NCNT + q
                    pltpu.async_copy(ones_v, cacc.at[dst_v.at[jq]], csem,
                                     add=True)
            return 0

        lax.fori_loop(0, CPT // NB, body, 0)
        for j in range(CPT + PF - NB, CPT):
            wait_scatter(j, j % NB)
        if with_count:
            def cdrain(j, _):
                pltpu.make_async_copy(ones_v, cacc.at[dst_v.at[j]],
                                      csem).wait()
                return 0
            lax.fori_loop(0, CPT_CNT, cdrain, 0)
        plsc.subcore_barrier()

        pltpu.sync_copy(acc.at[pl.ds(s * ROWS_PT, ROWS_PT)],
                        out_hbm.at[c, pl.ds(s * ROWS_PT, ROWS_PT)])
        if with_count:
            pltpu.sync_copy(cacc.at[pl.ds(s * ROWS_PT, ROWS_PT)],
                            cnt_hbm.at[c, pl.ds(s * ROWS_PT, ROWS_PT)])

    return sc_agg


_sc_aggs = {}


def _get_sc_agg(with_count):
    if with_count not in _sc_aggs:
        _sc_aggs[with_count] = _make_sc_agg(with_count)
    return _sc_aggs[with_count]


def _make_sc_cnt():
    mesh = plsc.VectorSubcoreMesh(
        core_axis_name="c", subcore_axis_name="s",
        num_cores=NC, num_subcores=NS)

    @functools.partial(
        pl.kernel,
        out_type=jax.ShapeDtypeStruct((NC, NPAD, L), jnp.int32),
        mesh=mesh,
        compiler_params=pltpu.CompilerParams(use_tc_tiling_on_sc=False),
        scratch_types=[
            pltpu.VMEM((CPT_CNT, CHUNK), jnp.int32),
            pltpu.VMEM((CHUNK, L), jnp.int32),
            pltpu.VMEM((CHUNK, L), jnp.int32),
            pltpu.VMEM_SHARED((NPAD, L), jnp.int32),
        ],
    )
    def sc_cnt(dsts_hbm, out_hbm, dst_v, ones_v, zbuf, acc):
        c = lax.axis_index("c")
        s = lax.axis_index("s")

        pltpu.sync_copy(dsts_hbm.at[s, pl.ds(c * CPT_CNT, CPT_CNT)], dst_v)

        ones = jnp.ones((L,), jnp.int32)
        zeros = jnp.zeros((L,), jnp.int32)

        def fill(r, _):
            ones_v[r, pl.ds(0, L)] = ones
            zbuf[r, pl.ds(0, L)] = zeros
            return 0

        lax.fori_loop(0, CHUNK, fill, 0)
        for r in range(ROWS_PT // CHUNK + 1):
            n = min(CHUNK, ROWS_PT - r * CHUNK)
            if n > 0:
                pltpu.sync_copy(
                    zbuf.at[pl.ds(0, n)],
                    acc.at[pl.ds(s * ROWS_PT + r * CHUNK, n)])
        plsc.subcore_barrier()

        def body(j, _):
            pltpu.sync_copy(ones_v, acc.at[dst_v.at[j]], add=True)
            return 0

        lax.fori_loop(0, CPT_CNT, body, 0)
        plsc.subcore_barrier()

        pltpu.sync_copy(acc.at[pl.ds(s * ROWS_PT, ROWS_PT)],
                        out_hbm.at[c, pl.ds(s * ROWS_PT, ROWS_PT)])

    return sc_cnt


_sc_cnt = None


def _get_sc_cnt():
    global _sc_cnt
    if _sc_cnt is None:
        _sc_cnt = _make_sc_cnt()
    return _sc_cnt


def _embed_body(x_ref, w_ref, b_ref, o_ref, ob_ref):
    h = (jnp.dot(x_ref[...], w_ref[...], preferred_element_type=jnp.float32)
         + b_ref[...])
    o_ref[...] = h
    ob_ref[...] = h.astype(jnp.bfloat16)


def _tc_embed(x, w, b2d):
    return pl.pallas_call(
        _embed_body,
        out_shape=[jax.ShapeDtypeStruct((N_NODES, D), jnp.float32),
                   jax.ShapeDtypeStruct((N_NODES, D), jnp.bfloat16)],
        grid=(N_NODES // BM,),
        in_specs=[
            pl.BlockSpec((BM, D), lambda i: (i, 0)),
            pl.BlockSpec((D, D), lambda i: (0, 0)),
            pl.BlockSpec((1, D), lambda i: (0, 0)),
        ],
        out_specs=[pl.BlockSpec((BM, D), lambda i: (i, 0)),
                   pl.BlockSpec((BM, D), lambda i: (i, 0))],
    )(x, w, b2d)


def _mfconv_core(p0_ref, p1_ref, x_ref, c0_ref, c1_ref,
                 wl_ref, wr_ref, bs_ref, relu):
    agg = jnp.concatenate([p0_ref[...], p1_ref[...]],
                          axis=1).astype(jnp.float32)
    x = x_ref[...]
    deg = jnp.minimum(c0_ref[...][:, 0:1] + c1_ref[...][:, 0:1], MAX_DEG)
    iot = lax.broadcasted_iota(jnp.int32, (BM, MAX_DEG + 1), 1)
    onehot = deg == iot
    out = jnp.dot(onehot.astype(jnp.float32), bs_ref[...],
                  preferred_element_type=jnp.float32)
    for i in range(MAX_DEG + 1):
        t = (jnp.dot(agg, wl_ref[i], preferred_element_type=jnp.float32)
             + jnp.dot(x, wr_ref[i], preferred_element_type=jnp.float32))
        out = out + jnp.where(onehot[:, i:i + 1], t, 0.0)
    if relu:
        out = jnp.maximum(out, 0.0)
    return out


def _mfconv_body(p0_ref, p1_ref, x_ref, c0_ref, c1_ref,
                 wl_ref, wr_ref, bs_ref, o_ref, ob_ref):
    out = _mfconv_core(p0_ref, p1_ref, x_ref, c0_ref, c1_ref,
                       wl_ref, wr_ref, bs_ref, relu=True)
    o_ref[...] = out
    ob_ref[...] = out.astype(jnp.bfloat16)


_MF_IN_SPECS = [
    pl.BlockSpec((BM, HD), lambda i: (i, 0)),
    pl.BlockSpec((BM, HD), lambda i: (i, 0)),
    pl.BlockSpec((BM, D), lambda i: (i, 0)),
    pl.BlockSpec((BM, L), lambda i: (i, 0)),
    pl.BlockSpec((BM, L), lambda i: (i, 0)),
    pl.BlockSpec((MAX_DEG + 1, D, D), lambda i: (0, 0, 0)),
    pl.BlockSpec((MAX_DEG + 1, D, D), lambda i: (0, 0, 0)),
    pl.BlockSpec((MAX_DEG + 1, D), lambda i: (0, 0)),
]


def _tc_mfconv(p0, p1, x, c0, c1, wl, wr, bsum):
    return pl.pallas_call(
        _mfconv_body,
        out_shape=[jax.ShapeDtypeStruct((N_NODES, D), jnp.float32),
                   jax.ShapeDtypeStruct((N_NODES, D), jnp.bfloat16)],
        grid=(N_NODES // BM,),
        in_specs=_MF_IN_SPECS,
        out_specs=[pl.BlockSpec((BM, D), lambda i: (i, 0)),
                   pl.BlockSpec((BM, D), lambda i: (i, 0))],
    )(p0, p1, x, c0, c1, wl, wr, bsum)


def _mfconv_pool_body(p0_ref, p1_ref, x_ref, c0_ref, c1_ref,
                      wl_ref, wr_ref, bs_ref, b_ref,
                      w1_ref, b1_ref, w2_ref, b2_ref, o_ref,
                      sums, counts):
    i = pl.program_id(0)
    nsteps = pl.num_programs(0)

    @pl.when(i == 0)
    def _():
        sums[...] = jnp.zeros_like(sums)
        counts[...] = jnp.zeros_like(counts)

    h = _mfconv_core(p0_ref, p1_ref, x_ref, c0_ref, c1_ref,
                     wl_ref, wr_ref, bs_ref, relu=False)
    bi = b_ref[...][:, 0:1]
    iot = lax.broadcasted_iota(jnp.int32, (BM, N_GRAPHS), 1)
    onehot = (bi == iot).astype(jnp.float32)
    sums[...] += jax.lax.dot_general(
        onehot, h, (((0,), (0,)), ((), ())),
        preferred_element_type=jnp.float32)
    counts[...] += jax.lax.dot_general(
        onehot, jnp.ones((BM, D), jnp.float32), (((0,), (0,)), ((), ())),
        preferred_element_type=jnp.float32)

    @pl.when(i == nsteps - 1)
    def _():
        pooled = sums[...] / jnp.maximum(counts[...], 1.0)
        z = jnp.maximum(
            jnp.dot(pooled, w1_ref[...], preferred_element_type=jnp.float32)
            + b1_ref[...], 0.0)
        o_ref[...] = (jnp.dot(z, w2_ref[...],
                              preferred_element_type=jnp.float32)
                      + b2_ref[...])


def _tc_mfconv_pool(p0, p1, x, c0, c1, wl, wr, bsum,
                    batch2d, w1, b1_2d, w2_pad, b2_2d):
    return pl.pallas_call(
        _mfconv_pool_body,
        out_shape=jax.ShapeDtypeStruct((N_GRAPHS, D), jnp.float32),
        grid=(N_NODES // BM,),
        in_specs=_MF_IN_SPECS + [
            pl.BlockSpec((BM, 1), lambda i: (i, 0)),
            pl.BlockSpec((D, D), lambda i: (0, 0)),
            pl.BlockSpec((1, D), lambda i: (0, 0)),
            pl.BlockSpec((D, D), lambda i: (0, 0)),
            pl.BlockSpec((1, D), lambda i: (0, 0)),
        ],
        out_specs=pl.BlockSpec((N_GRAPHS, D), lambda i: (0, 0)),
        scratch_shapes=[
            pltpu.VMEM((N_GRAPHS, D), jnp.float32),
            pltpu.VMEM((N_GRAPHS, D), jnp.float32),
        ],
    )(p0, p1, x, c0, c1, wl, wr, bsum,
      batch2d, w1, b1_2d, w2_pad, b2_2d)


def kernel(x, edge_index, edge_attr, batch_idx, embed_W, embed_b,
           Wl1, bl1, Wr1, br1, Wl2, bl2, Wr2, br2,
           lin1_W, lin1_b, lin2_W, lin2_b):
    del edge_attr

    src = edge_index[0].astype(jnp.int32)
    dst = edge_index[1].astype(jnp.int32)
    pad = EDGES_PAD - N_EDGES
    src2 = jnp.pad(2 * src, (0, pad)).reshape(NS, CPT, CHUNK)
    srcs2 = jnp.stack([src2, src2 + 1])
    dsts = jnp.pad(dst, (0, pad), constant_values=N_NODES).reshape(
        NS, CPT, CHUNK)

    cnt = _get_sc_cnt()(dsts)
    c0 = cnt[0, :N_NODES]
    c1 = cnt[1, :N_NODES]

    h0, h0_bf = _tc_embed(x, embed_W, embed_b.reshape(1, D))

    (p,) = _get_sc_agg(False)(h0_bf.reshape(2 * N_NODES, HD), srcs2, dsts)
    g1, g1_bf = _tc_mfconv(p[0, :N_NODES], p[1, :N_NODES], h0, c0, c1,
                           Wl1, Wr1, bl1 + br1)

    (p2,) = _get_sc_agg(False)(g1_bf.reshape(2 * N_NODES, HD), srcs2, dsts)
    w2_pad = jnp.pad(lin2_W, ((0, 0), (0, D - 1)))
    b2_2d = jnp.pad(lin2_b.reshape(1, 1), ((0, 0), (0, D - 1)))
    res = _tc_mfconv_pool(p2[0, :N_NODES], p2[1, :N_NODES], g1, c0, c1,
                          Wl2, Wr2, bl2 + br2,
                          batch_idx.astype(jnp.int32).reshape(N_NODES, 1),
                          lin1_W, lin1_b.reshape(1, D), w2_pad, b2_2d)
    return res[:, 0:1]
